# baseline (speedup 1.0000x reference)
import functools

import jax
import jax.numpy as jnp
from jax import lax
from jax.experimental import pallas as pl
from jax.experimental.pallas import tpu as pltpu
from jax.experimental.pallas import tpu_sc as plsc

MAX_WEEKS = 160
EMBED_DIM = 64
BATCH = 16384
HIST = 200

N = BATCH * HIST
NC, NS = 2, 16
NW = NC * NS
BATCH_PER_W = BATCH // NW
DESC = 100
IDX_PAD = 128
K = 4
ROWS_PER = K * HIST // DESC
N_ITER = BATCH_PER_W // K
NBUF = 2
N_OUTER = N_ITER // NBUF
N_IDX_ROWS = N // DESC

_mesh = plsc.VectorSubcoreMesh(core_axis_name="c", subcore_axis_name="s")


@functools.partial(
    pl.kernel,
    out_type=jax.ShapeDtypeStruct((BATCH, HIST, EMBED_DIM), jnp.float32),
    mesh=_mesh,
    scratch_types=[
        pltpu.VMEM((NBUF, ROWS_PER, IDX_PAD), jnp.int32),
        pltpu.VMEM((NBUF, K, HIST, EMBED_DIM), jnp.float32),
        pltpu.VMEM_SHARED((MAX_WEEKS, EMBED_DIM), jnp.float32),
        pltpu.SemaphoreType.DMA,
        pltpu.SemaphoreType.DMA,
        pltpu.SemaphoreType.DMA,
        pltpu.SemaphoreType.DMA,
    ],
    compiler_params=pltpu.CompilerParams(use_tc_tiling_on_sc=False),
)
def _gather_kernel(idx_hbm, table_hbm, out_hbm, idx_v, rows_v, table_v,
                   isem, gsem, osem_a, osem_b):
    wid = lax.axis_index("s") * NC + lax.axis_index("c")

    @pl.when(lax.axis_index("s") == 0)
    def _stage_table():
        pltpu.sync_copy(table_hbm, table_v)

    plsc.subcore_barrier()

    base_irow = wid * (BATCH_PER_W * HIST // DESC)
    base_batch = wid * BATCH_PER_W
    osems = [osem_a, osem_b]

    pltpu.async_copy(idx_hbm.at[pl.ds(base_irow, ROWS_PER)], idx_v.at[0], isem)

    def outer(o, carry):
        for b in range(NBUF):
            t = NBUF * o + b
            pltpu.make_async_copy(
                idx_hbm.at[pl.ds(0, ROWS_PER)], idx_v.at[b], isem
            ).wait()

            @pl.when(t + 1 < N_ITER)
            def _prefetch():
                irow = base_irow + (t + 1) * ROWS_PER
                pltpu.async_copy(
                    idx_hbm.at[pl.ds(irow, ROWS_PER)], idx_v.at[1 - b], isem
                )

            @pl.when(t >= NBUF)
            def _drain_prev_out():
                pltpu.make_async_copy(
                    rows_v.at[b], out_hbm.at[pl.ds(0, K)], osems[b]
                ).wait()

            handles = [
                pltpu.async_copy(
                    table_v.at[idx_v.at[b].at[j].at[pl.ds(0, DESC)]],
                    rows_v.at[b].at[j // 2].at[pl.ds((j % 2) * DESC, DESC)],
                    gsem,
                )
                for j in range(ROWS_PER)
            ]
            for h in handles:
                h.wait()

            pltpu.async_copy(
                rows_v.at[b],
                out_hbm.at[pl.ds(base_batch + t * K, K)],
                osems[b],
            )
        return carry

    lax.fori_loop(0, N_OUTER, outer, 0)

    for b in range(NBUF):
        pltpu.make_async_copy(
            rows_v.at[b], out_hbm.at[pl.ds(0, K)], osems[b]
        ).wait()


def kernel(week_numbers, week_embed):
    idx = week_numbers.reshape(N_IDX_ROWS, DESC).astype(jnp.int32)
    idx = jnp.pad(idx, ((0, 0), (0, IDX_PAD - DESC)))
    return _gather_kernel(idx, week_embed)


# speedup vs baseline: 2.1482x; 2.1482x over previous
import functools

import jax
import jax.numpy as jnp
from jax import lax
from jax.experimental import pallas as pl
from jax.experimental.pallas import tpu as pltpu
from jax.experimental.pallas import tpu_sc as plsc

MAX_WEEKS = 160
EMBED_DIM = 64
BATCH = 16384
HIST = 200

N = BATCH * HIST
NC, NS = 2, 16
NW = NC * NS
BATCH_PER_W = BATCH // NW
DESC = 100
IDX_PAD = 128
K = 4
ROWS_PER = K * HIST // DESC
N_ITER = BATCH_PER_W // K
NBUF = 2
N_OUTER = N_ITER // NBUF
N_IDX_ROWS = N // DESC

_mesh = plsc.VectorSubcoreMesh(core_axis_name="c", subcore_axis_name="s")


@functools.partial(
    pl.kernel,
    out_type=jax.ShapeDtypeStruct((BATCH, HIST, 2 * EMBED_DIM), jnp.float32),
    mesh=_mesh,
    scratch_types=[
        pltpu.VMEM((NBUF, ROWS_PER, IDX_PAD), jnp.int32),
        pltpu.VMEM((NBUF, K, HIST, EMBED_DIM), jnp.float32),
        pltpu.VMEM_SHARED((MAX_WEEKS, EMBED_DIM), jnp.float32),
        pltpu.SemaphoreType.DMA,
        pltpu.SemaphoreType.DMA,
        pltpu.SemaphoreType.DMA,
        pltpu.SemaphoreType.DMA,
    ],
    compiler_params=pltpu.CompilerParams(use_tc_tiling_on_sc=False),
)
def _gather_kernel(idx_hbm, table_hbm, out_hbm, idx_v, rows_v, table_v,
                   isem, gsem, osem_a, osem_b):
    wid = lax.axis_index("s") * NC + lax.axis_index("c")

    @pl.when(lax.axis_index("s") == 0)
    def _stage_table():
        pltpu.sync_copy(table_hbm, table_v)

    plsc.subcore_barrier()

    base_irow = wid * (BATCH_PER_W * HIST // DESC)
    base_batch = wid * BATCH_PER_W
    osems = [osem_a, osem_b]

    pltpu.async_copy(idx_hbm.at[pl.ds(base_irow, ROWS_PER)], idx_v.at[0], isem)

    def outer(o, carry):
        for b in range(NBUF):
            t = NBUF * o + b
            pltpu.make_async_copy(
                idx_hbm.at[pl.ds(0, ROWS_PER)], idx_v.at[b], isem
            ).wait()

            @pl.when(t + 1 < N_ITER)
            def _prefetch():
                irow = base_irow + (t + 1) * ROWS_PER
                pltpu.async_copy(
                    idx_hbm.at[pl.ds(irow, ROWS_PER)], idx_v.at[1 - b], isem
                )

            @pl.when(t >= NBUF)
            def _drain_prev_out():
                pltpu.make_async_copy(
                    rows_v.at[b],
                    out_hbm.at[pl.ds(0, K), :, pl.ds(0, EMBED_DIM)],
                    osems[b],
                ).wait()

            handles = [
                pltpu.async_copy(
                    table_v.at[idx_v.at[b].at[j].at[pl.ds(0, DESC)]],
                    rows_v.at[b].at[j // 2].at[pl.ds((j % 2) * DESC, DESC)],
                    gsem,
                )
                for j in range(ROWS_PER)
            ]
            for h in handles:
                h.wait()

            pltpu.async_copy(
                rows_v.at[b],
                out_hbm.at[
                    pl.ds(base_batch + t * K, K), :, pl.ds(0, EMBED_DIM)
                ],
                osems[b],
            )
        return carry

    lax.fori_loop(0, N_OUTER, outer, 0)

    for b in range(NBUF):
        pltpu.make_async_copy(
            rows_v.at[b],
            out_hbm.at[pl.ds(0, K), :, pl.ds(0, EMBED_DIM)],
            osems[b],
        ).wait()


def kernel(week_numbers, week_embed):
    idx = week_numbers.reshape(N_IDX_ROWS, DESC).astype(jnp.int32)
    idx = jnp.pad(idx, ((0, 0), (0, IDX_PAD - DESC)))
    out = _gather_kernel(idx, week_embed)
    return out[:, :, :EMBED_DIM]
